# baseline (speedup 1.0000x reference)
import functools

import jax
import jax.numpy as jnp
from jax import lax
from jax.experimental import pallas as pl
from jax.experimental.pallas import tpu as pltpu
from jax.experimental.pallas import tpu_sc as plsc

_EPS = 1e-12
_BR = 256
_C = 32
_NBUF = 3


@functools.lru_cache(maxsize=None)
def _make_sc_gather(V, D, B):
    info = plsc.get_sparse_core_info()
    NC, NS = info.num_cores, info.num_subcores
    NW = NC * NS
    assert B % (8 * NW) == 0
    b_per_w = B // NW
    C, NBUF = _C, _NBUF
    assert b_per_w % C == 0
    NCH = b_per_w // C
    assert NCH >= NBUF
    mesh = plsc.VectorSubcoreMesh(core_axis_name="c", subcore_axis_name="s")

    @functools.partial(
        pl.kernel,
        mesh=mesh,
        out_type=jax.ShapeDtypeStruct((B, D), jnp.float32),
        scratch_types=[pltpu.VMEM((NCH, C), jnp.int32)]
        + [pltpu.VMEM((C, D), jnp.float32) for _ in range(NBUF)]
        + [pltpu.SemaphoreType.DMA for _ in range(2 * NBUF)],
    )
    def gather_k(table_hbm, idx_hbm, out_hbm, idx_v, *rest):
        bufs = rest[:NBUF]
        gs = rest[NBUF:2 * NBUF]
        os = rest[2 * NBUF:]
        wid = lax.axis_index("s") * NC + lax.axis_index("c")
        base = wid * b_per_w
        for c in range(NCH):
            pltpu.sync_copy(idx_hbm.at[pl.ds(base + c * C, C)], idx_v.at[c])
        g = [None] * NCH
        o = [None] * NCH
        for c in range(NBUF - 1):
            g[c] = pltpu.async_copy(
                table_hbm.at[idx_v.at[c]], bufs[c % NBUF], gs[c % NBUF])
        for c in range(NCH):
            nxt = c + NBUF - 1
            if nxt < NCH:
                if c >= 1:
                    o[c - 1].wait()
                g[nxt] = pltpu.async_copy(
                    table_hbm.at[idx_v.at[nxt]], bufs[nxt % NBUF],
                    gs[nxt % NBUF])
            g[c].wait()
            o[c] = pltpu.async_copy(
                bufs[c % NBUF], out_hbm.at[pl.ds(base + c * C, C)],
                os[c % NBUF])
        for c in range(NCH - NBUF, NCH):
            o[c].wait()

    return gather_k


def _ln_body(x_ref, tt_ref, tp_ref, g_ref, b_ref, o_ref):
    x = x_ref[...]
    t0 = tp_ref[0:1, :]
    t1 = tp_ref[1:2, :]
    f = tt_ref[...]
    x = x + t0 + f * (t1 - t0)
    d = x.shape[-1]
    s1 = jnp.sum(x, axis=-1, keepdims=True)
    s2 = jnp.sum(x * x, axis=-1, keepdims=True)
    mean = s1 * (1.0 / d)
    var = s2 * (1.0 / d) - mean * mean
    k = lax.rsqrt(var + _EPS)
    scale = k * g_ref[...]
    shift = b_ref[...] - mean * scale
    o_ref[...] = x * scale + shift


@functools.lru_cache(maxsize=None)
def _make_tc_ln(B, D):
    assert B % _BR == 0
    grid = (B // _BR,)
    return pl.pallas_call(
        _ln_body,
        grid=grid,
        in_specs=[
            pl.BlockSpec((_BR, D), lambda i: (i, 0)),
            pl.BlockSpec((_BR, 1), lambda i: (i, 0)),
            pl.BlockSpec((2, D), lambda i: (0, 0)),
            pl.BlockSpec((1, D), lambda i: (0, 0)),
            pl.BlockSpec((1, D), lambda i: (0, 0)),
        ],
        out_specs=pl.BlockSpec((_BR, D), lambda i: (i, 0)),
        out_shape=jax.ShapeDtypeStruct((B, D), jnp.float32),
    )


def kernel(input_ids, token_type_ids, word_table, type_table, gamma, beta):
    NB, S = input_ids.shape
    V, D = word_table.shape
    Bt = NB * S
    ids = input_ids.reshape(-1).astype(jnp.int32)
    ttf = token_type_ids.reshape(-1, 1).astype(jnp.float32)
    gathered = _make_sc_gather(V, D, Bt)(word_table, ids)
    out = _make_tc_ln(Bt, D)(
        gathered, ttf, type_table, gamma.reshape(1, -1), beta.reshape(1, -1))
    return out.reshape(NB, S, D)


# speedup vs baseline: 1.1183x; 1.1183x over previous
import functools

import jax
import jax.numpy as jnp
from jax import lax
from jax.experimental import pallas as pl
from jax.experimental.pallas import tpu as pltpu
from jax.experimental.pallas import tpu_sc as plsc

_EPS = 1e-12
_BR = 512
_C = 32
_NBUF = 3


@functools.lru_cache(maxsize=None)
def _make_sc_gather(V, D, B):
    info = plsc.get_sparse_core_info()
    NC, NS = info.num_cores, info.num_subcores
    NW = NC * NS
    assert B % (8 * NW) == 0
    b_per_w = B // NW
    C, NBUF = _C, _NBUF
    assert b_per_w % C == 0
    NCH = b_per_w // C
    assert NCH >= NBUF
    mesh = plsc.VectorSubcoreMesh(core_axis_name="c", subcore_axis_name="s")

    @functools.partial(
        pl.kernel,
        mesh=mesh,
        out_type=jax.ShapeDtypeStruct((B, D), jnp.float32),
        scratch_types=[pltpu.VMEM((NCH, C), jnp.int32)]
        + [pltpu.VMEM((C, D), jnp.float32) for _ in range(NBUF)]
        + [pltpu.SemaphoreType.DMA for _ in range(2 * NBUF)],
    )
    def gather_k(table_hbm, idx_hbm, out_hbm, idx_v, *rest):
        bufs = rest[:NBUF]
        gs = rest[NBUF:2 * NBUF]
        os = rest[2 * NBUF:]
        wid = lax.axis_index("s") * NC + lax.axis_index("c")
        base = wid * b_per_w
        for c in range(NCH):
            pltpu.sync_copy(idx_hbm.at[pl.ds(base + c * C, C)], idx_v.at[c])
        g = [None] * NCH
        o = [None] * NCH
        for c in range(NBUF - 1):
            g[c] = pltpu.async_copy(
                table_hbm.at[idx_v.at[c]], bufs[c % NBUF], gs[c % NBUF])
        for c in range(NCH):
            nxt = c + NBUF - 1
            if nxt < NCH:
                if c >= 1:
                    o[c - 1].wait()
                g[nxt] = pltpu.async_copy(
                    table_hbm.at[idx_v.at[nxt]], bufs[nxt % NBUF],
                    gs[nxt % NBUF])
            g[c].wait()
            o[c] = pltpu.async_copy(
                bufs[c % NBUF], out_hbm.at[pl.ds(base + c * C, C)],
                os[c % NBUF])
        for c in range(NCH - NBUF, NCH):
            o[c].wait()

    return gather_k


def _ln_body(x_ref, tt_ref, tp_ref, g_ref, b_ref, o_ref):
    x = x_ref[...]
    t0 = tp_ref[0:1, :]
    t1 = tp_ref[1:2, :]
    f = tt_ref[...]
    x = x + t0 + f * (t1 - t0)
    d = x.shape[-1]
    s1 = jnp.sum(x, axis=-1, keepdims=True)
    s2 = jnp.sum(x * x, axis=-1, keepdims=True)
    mean = s1 * (1.0 / d)
    var = s2 * (1.0 / d) - mean * mean
    k = lax.rsqrt(var + _EPS)
    scale = k * g_ref[...]
    shift = b_ref[...] - mean * scale
    o_ref[...] = x * scale + shift


@functools.lru_cache(maxsize=None)
def _make_tc_ln(B, D):
    assert B % _BR == 0
    grid = (B // _BR,)
    return pl.pallas_call(
        _ln_body,
        grid=grid,
        in_specs=[
            pl.BlockSpec((_BR, D), lambda i: (i, 0)),
            pl.BlockSpec((_BR, 1), lambda i: (i, 0)),
            pl.BlockSpec((2, D), lambda i: (0, 0)),
            pl.BlockSpec((1, D), lambda i: (0, 0)),
            pl.BlockSpec((1, D), lambda i: (0, 0)),
        ],
        out_specs=pl.BlockSpec((_BR, D), lambda i: (i, 0)),
        out_shape=jax.ShapeDtypeStruct((B, D), jnp.float32),
    )


def kernel(input_ids, token_type_ids, word_table, type_table, gamma, beta):
    NB, S = input_ids.shape
    V, D = word_table.shape
    Bt = NB * S
    ids = input_ids.reshape(-1).astype(jnp.int32)
    ttf = token_type_ids.reshape(-1, 1).astype(jnp.float32)
    gathered = _make_sc_gather(V, D, Bt)(word_table, ids)
    out = _make_tc_ln(Bt, D)(
        gathered, ttf, type_table, gamma.reshape(1, -1), beta.reshape(1, -1))
    return out.reshape(NB, S, D)


# speedup vs baseline: 1.1715x; 1.0476x over previous
import functools

import jax
import jax.numpy as jnp
from jax import lax
from jax.experimental import pallas as pl
from jax.experimental.pallas import tpu as pltpu
from jax.experimental.pallas import tpu_sc as plsc

_EPS = 1e-12
_BR = 1024
_C = 32
_NBUF = 3


@functools.lru_cache(maxsize=None)
def _make_sc_gather(V, D, B):
    info = plsc.get_sparse_core_info()
    NC, NS = info.num_cores, info.num_subcores
    NW = NC * NS
    assert B % (8 * NW) == 0
    b_per_w = B // NW
    C, NBUF = _C, _NBUF
    assert b_per_w % C == 0
    NCH = b_per_w // C
    assert NCH >= NBUF
    mesh = plsc.VectorSubcoreMesh(core_axis_name="c", subcore_axis_name="s")

    @functools.partial(
        pl.kernel,
        mesh=mesh,
        out_type=jax.ShapeDtypeStruct((B, D), jnp.float32),
        scratch_types=[pltpu.VMEM((NCH, C), jnp.int32)]
        + [pltpu.VMEM((C, D), jnp.float32) for _ in range(NBUF)]
        + [pltpu.SemaphoreType.DMA for _ in range(2 * NBUF)],
    )
    def gather_k(table_hbm, idx_hbm, out_hbm, idx_v, *rest):
        bufs = rest[:NBUF]
        gs = rest[NBUF:2 * NBUF]
        os = rest[2 * NBUF:]
        wid = lax.axis_index("s") * NC + lax.axis_index("c")
        base = wid * b_per_w
        for c in range(NCH):
            pltpu.sync_copy(idx_hbm.at[pl.ds(base + c * C, C)], idx_v.at[c])
        g = [None] * NCH
        o = [None] * NCH
        for c in range(NBUF - 1):
            g[c] = pltpu.async_copy(
                table_hbm.at[idx_v.at[c]], bufs[c % NBUF], gs[c % NBUF])
        for c in range(NCH):
            nxt = c + NBUF - 1
            if nxt < NCH:
                if c >= 1:
                    o[c - 1].wait()
                g[nxt] = pltpu.async_copy(
                    table_hbm.at[idx_v.at[nxt]], bufs[nxt % NBUF],
                    gs[nxt % NBUF])
            g[c].wait()
            o[c] = pltpu.async_copy(
                bufs[c % NBUF], out_hbm.at[pl.ds(base + c * C, C)],
                os[c % NBUF])
        for c in range(NCH - NBUF, NCH):
            o[c].wait()

    return gather_k


def _ln_body(x_ref, tt_ref, tp_ref, g_ref, b_ref, o_ref):
    x = x_ref[...]
    t0 = tp_ref[0:1, :]
    t1 = tp_ref[1:2, :]
    f = tt_ref[...]
    x = x + t0 + f * (t1 - t0)
    d = x.shape[-1]
    s1 = jnp.sum(x, axis=-1, keepdims=True)
    s2 = jnp.sum(x * x, axis=-1, keepdims=True)
    mean = s1 * (1.0 / d)
    var = s2 * (1.0 / d) - mean * mean
    k = lax.rsqrt(var + _EPS)
    scale = k * g_ref[...]
    shift = b_ref[...] - mean * scale
    o_ref[...] = x * scale + shift


@functools.lru_cache(maxsize=None)
def _make_tc_ln(B, D):
    assert B % _BR == 0
    grid = (B // _BR,)
    return pl.pallas_call(
        _ln_body,
        grid=grid,
        in_specs=[
            pl.BlockSpec((_BR, D), lambda i: (i, 0)),
            pl.BlockSpec((_BR, 1), lambda i: (i, 0)),
            pl.BlockSpec((2, D), lambda i: (0, 0)),
            pl.BlockSpec((1, D), lambda i: (0, 0)),
            pl.BlockSpec((1, D), lambda i: (0, 0)),
        ],
        out_specs=pl.BlockSpec((_BR, D), lambda i: (i, 0)),
        out_shape=jax.ShapeDtypeStruct((B, D), jnp.float32),
    )


def kernel(input_ids, token_type_ids, word_table, type_table, gamma, beta):
    NB, S = input_ids.shape
    V, D = word_table.shape
    Bt = NB * S
    ids = input_ids.reshape(-1).astype(jnp.int32)
    ttf = token_type_ids.reshape(-1, 1).astype(jnp.float32)
    gathered = _make_sc_gather(V, D, Bt)(word_table, ids)
    out = _make_tc_ln(Bt, D)(
        gathered, ttf, type_table, gamma.reshape(1, -1), beta.reshape(1, -1))
    return out.reshape(NB, S, D)


# speedup vs baseline: 1.1784x; 1.0059x over previous
import functools

import jax
import jax.numpy as jnp
from jax import lax
from jax.experimental import pallas as pl
from jax.experimental.pallas import tpu as pltpu
from jax.experimental.pallas import tpu_sc as plsc

_EPS = 1e-12
_BR = 2048
_C = 32
_NBUF = 3


@functools.lru_cache(maxsize=None)
def _make_sc_gather(V, D, B):
    info = plsc.get_sparse_core_info()
    NC, NS = info.num_cores, info.num_subcores
    NW = NC * NS
    assert B % (8 * NW) == 0
    b_per_w = B // NW
    C, NBUF = _C, _NBUF
    assert b_per_w % C == 0
    NCH = b_per_w // C
    assert NCH >= NBUF
    mesh = plsc.VectorSubcoreMesh(core_axis_name="c", subcore_axis_name="s")

    @functools.partial(
        pl.kernel,
        mesh=mesh,
        out_type=jax.ShapeDtypeStruct((B, D), jnp.float32),
        scratch_types=[pltpu.VMEM((NCH, C), jnp.int32)]
        + [pltpu.VMEM((C, D), jnp.float32) for _ in range(NBUF)]
        + [pltpu.SemaphoreType.DMA for _ in range(2 * NBUF)],
    )
    def gather_k(table_hbm, idx_hbm, out_hbm, idx_v, *rest):
        bufs = rest[:NBUF]
        gs = rest[NBUF:2 * NBUF]
        os = rest[2 * NBUF:]
        wid = lax.axis_index("s") * NC + lax.axis_index("c")
        base = wid * b_per_w
        for c in range(NCH):
            pltpu.sync_copy(idx_hbm.at[pl.ds(base + c * C, C)], idx_v.at[c])
        g = [None] * NCH
        o = [None] * NCH
        for c in range(NBUF - 1):
            g[c] = pltpu.async_copy(
                table_hbm.at[idx_v.at[c]], bufs[c % NBUF], gs[c % NBUF])
        for c in range(NCH):
            nxt = c + NBUF - 1
            if nxt < NCH:
                if c >= 1:
                    o[c - 1].wait()
                g[nxt] = pltpu.async_copy(
                    table_hbm.at[idx_v.at[nxt]], bufs[nxt % NBUF],
                    gs[nxt % NBUF])
            g[c].wait()
            o[c] = pltpu.async_copy(
                bufs[c % NBUF], out_hbm.at[pl.ds(base + c * C, C)],
                os[c % NBUF])
        for c in range(NCH - NBUF, NCH):
            o[c].wait()

    return gather_k


def _ln_body(x_ref, tt_ref, tp_ref, g_ref, b_ref, o_ref):
    x = x_ref[...]
    t0 = tp_ref[0:1, :]
    t1 = tp_ref[1:2, :]
    f = tt_ref[...]
    x = x + t0 + f * (t1 - t0)
    d = x.shape[-1]
    s1 = jnp.sum(x, axis=-1, keepdims=True)
    s2 = jnp.sum(x * x, axis=-1, keepdims=True)
    mean = s1 * (1.0 / d)
    var = s2 * (1.0 / d) - mean * mean
    k = lax.rsqrt(var + _EPS)
    scale = k * g_ref[...]
    shift = b_ref[...] - mean * scale
    o_ref[...] = x * scale + shift


@functools.lru_cache(maxsize=None)
def _make_tc_ln(B, D):
    assert B % _BR == 0
    grid = (B // _BR,)
    return pl.pallas_call(
        _ln_body,
        grid=grid,
        in_specs=[
            pl.BlockSpec((_BR, D), lambda i: (i, 0)),
            pl.BlockSpec((_BR, 1), lambda i: (i, 0)),
            pl.BlockSpec((2, D), lambda i: (0, 0)),
            pl.BlockSpec((1, D), lambda i: (0, 0)),
            pl.BlockSpec((1, D), lambda i: (0, 0)),
        ],
        out_specs=pl.BlockSpec((_BR, D), lambda i: (i, 0)),
        out_shape=jax.ShapeDtypeStruct((B, D), jnp.float32),
    )


def kernel(input_ids, token_type_ids, word_table, type_table, gamma, beta):
    NB, S = input_ids.shape
    V, D = word_table.shape
    Bt = NB * S
    ids = input_ids.reshape(-1).astype(jnp.int32)
    ttf = token_type_ids.reshape(-1, 1).astype(jnp.float32)
    gathered = _make_sc_gather(V, D, Bt)(word_table, ids)
    out = _make_tc_ln(Bt, D)(
        gathered, ttf, type_table, gamma.reshape(1, -1), beta.reshape(1, -1))
    return out.reshape(NB, S, D)


# speedup vs baseline: 1.2171x; 1.0328x over previous
import functools

import jax
import jax.numpy as jnp
from jax import lax
from jax.experimental import pallas as pl
from jax.experimental.pallas import tpu as pltpu
from jax.experimental.pallas import tpu_sc as plsc

_EPS = 1e-12
_BR = 2048
_C = 32
_NBUF = 3
_NCHUNK = 2


@functools.lru_cache(maxsize=None)
def _make_sc_gather(V, D, B, off):
    info = plsc.get_sparse_core_info()
    NC, NS = info.num_cores, info.num_subcores
    NW = NC * NS
    assert B % (8 * NW) == 0
    b_per_w = B // NW
    C, NBUF = _C, _NBUF
    assert b_per_w % C == 0
    NCH = b_per_w // C
    assert NCH >= NBUF
    mesh = plsc.VectorSubcoreMesh(core_axis_name="c", subcore_axis_name="s")

    @functools.partial(
        pl.kernel,
        mesh=mesh,
        out_type=jax.ShapeDtypeStruct((B, D), jnp.float32),
        scratch_types=[pltpu.VMEM((NCH, C), jnp.int32)]
        + [pltpu.VMEM((C, D), jnp.float32) for _ in range(NBUF)]
        + [pltpu.SemaphoreType.DMA for _ in range(2 * NBUF)],
    )
    def gather_k(table_hbm, idx_hbm, out_hbm, idx_v, *rest):
        bufs = rest[:NBUF]
        gs = rest[NBUF:2 * NBUF]
        os = rest[2 * NBUF:]
        wid = lax.axis_index("s") * NC + lax.axis_index("c")
        base = wid * b_per_w
        for c in range(NCH):
            pltpu.sync_copy(idx_hbm.at[pl.ds(off + base + c * C, C)],
                            idx_v.at[c])
        g = [None] * NCH
        o = [None] * NCH
        for c in range(NBUF - 1):
            g[c] = pltpu.async_copy(
                table_hbm.at[idx_v.at[c]], bufs[c % NBUF], gs[c % NBUF])
        for c in range(NCH):
            nxt = c + NBUF - 1
            if nxt < NCH:
                if c >= 1:
                    o[c - 1].wait()
                g[nxt] = pltpu.async_copy(
                    table_hbm.at[idx_v.at[nxt]], bufs[nxt % NBUF],
                    gs[nxt % NBUF])
            g[c].wait()
            o[c] = pltpu.async_copy(
                bufs[c % NBUF], out_hbm.at[pl.ds(base + c * C, C)],
                os[c % NBUF])
        for c in range(NCH - NBUF, NCH):
            o[c].wait()

    return gather_k


def _ln_core(x_ref, tt_ref, tp_ref, g_ref, b_ref, o_ref):
    x = x_ref[...]
    t0 = tp_ref[0:1, :]
    t1 = tp_ref[1:2, :]
    f = tt_ref[...]
    x = x + t0 + f * (t1 - t0)
    d = x.shape[-1]
    s1 = jnp.sum(x, axis=-1, keepdims=True)
    s2 = jnp.sum(x * x, axis=-1, keepdims=True)
    mean = s1 * (1.0 / d)
    var = s2 * (1.0 / d) - mean * mean
    k = lax.rsqrt(var + _EPS)
    scale = k * g_ref[...]
    shift = b_ref[...] - mean * scale
    o_ref[...] = x * scale + shift


def _ln_body_alias(buf_ref, x_ref, tt_ref, tp_ref, g_ref, b_ref, o_ref):
    del buf_ref
    _ln_core(x_ref, tt_ref, tp_ref, g_ref, b_ref, o_ref)


@functools.lru_cache(maxsize=None)
def _make_tc_ln(B_total, B_chunk, D, block_off, aliased):
    assert B_chunk % _BR == 0
    grid = (B_chunk // _BR,)
    data_specs = [
        pl.BlockSpec((_BR, D), lambda i: (i, 0)),
        pl.BlockSpec((_BR, 1), lambda i: (block_off + i, 0)),
        pl.BlockSpec((2, D), lambda i: (0, 0)),
        pl.BlockSpec((1, D), lambda i: (0, 0)),
        pl.BlockSpec((1, D), lambda i: (0, 0)),
    ]
    if aliased:
        in_specs = [pl.BlockSpec(memory_space=pl.ANY)] + data_specs
        body = _ln_body_alias
        aliases = {0: 0}
    else:
        in_specs = data_specs
        body = _ln_core
        aliases = {}
    return pl.pallas_call(
        body,
        grid=grid,
        in_specs=in_specs,
        out_specs=pl.BlockSpec((_BR, D), lambda i: (block_off + i, 0)),
        out_shape=jax.ShapeDtypeStruct((B_total, D), jnp.float32),
        input_output_aliases=aliases,
    )


def kernel(input_ids, token_type_ids, word_table, type_table, gamma, beta):
    NB, S = input_ids.shape
    V, D = word_table.shape
    Bt = NB * S
    Bc = Bt // _NCHUNK
    gamma2 = gamma.reshape(1, -1)
    beta2 = beta.reshape(1, -1)
    ids = input_ids.reshape(-1).astype(jnp.int32)
    ttf = token_type_ids.reshape(-1, 1).astype(jnp.float32)
    blocks_per_chunk = Bc // _BR

    gathered = [
        _make_sc_gather(V, D, Bc, b * Bc)(word_table, ids)
        for b in range(_NCHUNK)
    ]

    out = None
    for b in range(_NCHUNK):
        ln = _make_tc_ln(Bt, Bc, D, b * blocks_per_chunk, b > 0)
        if b == 0:
            out = ln(gathered[b], ttf, type_table, gamma2, beta2)
        else:
            out = ln(out, gathered[b], ttf, type_table, gamma2, beta2)
    return out.reshape(NB, S, D)
